# baseline (speedup 1.0000x reference)
import jax
import jax.numpy as jnp
from jax import lax
from jax.experimental import pallas as pl
from jax.experimental.pallas import tpu as pltpu
from jax.experimental.pallas import tpu_sc as plsc

_B = 16384
_S = 50
_V = 10000
_D = 128

_NC = 2
_NS = 16
_NW = _NC * _NS
_ROWS_PER_W = _B // _NW
_NB = 4
_CHUNKS = _ROWS_PER_W // _NB
_LANES = 16
_BL = 2 * _LANES
_GRP = 10


def _table_body(emb_ref, w_ref, b_ref, out_ref):
    j = pl.program_id(0)
    p = lax.dot_general(
        emb_ref[...], w_ref[...],
        dimension_numbers=(((1,), (1,)), ((), ())),
        preferred_element_type=jnp.float32,
    )
    sel = jnp.where(j == 0, 1.0, 0.0).astype(jnp.float32)
    p = p + b_ref[...] * sel
    out_ref[...] = p.astype(jnp.bfloat16)


def _build_table(emb, W, b2d):
    return pl.pallas_call(
        _table_body,
        grid=(_S,),
        in_specs=[
            pl.BlockSpec((_V, _D), lambda j: (0, 0)),
            pl.BlockSpec((_D, _D), lambda j: (0, j)),
            pl.BlockSpec((1, _D), lambda j: (0, 0)),
        ],
        out_specs=pl.BlockSpec((_V, _D), lambda j: (j, 0)),
        out_shape=jax.ShapeDtypeStruct((_S * _V, _D), jnp.bfloat16),
    )(emb, W, b2d)


def _gather_body(p_hbm, x_hbm, out_hbm, idx_all, rows_v, out_v, sem0, sem1):
    n_dma = _NB // 2
    rows_per_chunk = _NB * _S
    nd = _D // _BL
    wid = lax.axis_index("s") * _NC + lax.axis_index("c")
    row0 = wid * _ROWS_PER_W
    sems = (sem0, sem1)

    def start_gathers(chunk, slot):
        for g in range(n_dma):
            pltpu.async_copy(
                p_hbm.at[idx_all.at[chunk * n_dma + g]],
                rows_v.at[slot, pl.ds(g * 2 * _S, 2 * _S)],
                sems[slot],
            )

    def wait_gathers(slot):
        pltpu.make_async_copy(
            p_hbm.at[pl.ds(0, rows_per_chunk)], rows_v.at[slot],
            sems[slot],
        ).wait()

    def accumulate_and_store(chunk, slot):
        for r in range(_NB):
            base = r * _S
            facc = [
                jnp.zeros((_LANES,), jnp.float32)
                for _ in range(_D // _LANES)
            ]
            for grp in range(_S // _GRP):
                gbase = base + grp * _GRP

                def load_bf(j):
                    return tuple(
                        rows_v[slot, gbase + j, pl.ds(d * _BL, _BL)]
                        for d in range(nd)
                    )

                def jstep(j, acc):
                    row = load_bf(j)
                    return tuple(acc[d] + row[d] for d in range(nd))

                pacc = lax.fori_loop(1, _GRP, jstep, load_bf(0), unroll=3)
                for d in range(nd):
                    a, bvec = plsc.unpack(
                        pacc[d], format=plsc.PackFormat.INTERLEAVED
                    )
                    facc[2 * d] = facc[2 * d] + a
                    facc[2 * d + 1] = facc[2 * d + 1] + bvec
            for d in range(_D // _LANES):
                out_v[r, pl.ds(d * _LANES, _LANES)] = facc[d]
        pltpu.sync_copy(
            out_v, out_hbm.at[pl.ds(row0 + chunk * _NB, _NB)]
        )

    pltpu.sync_copy(
        x_hbm.at[pl.ds(wid * (_ROWS_PER_W // 2), _ROWS_PER_W // 2)],
        idx_all,
    )
    start_gathers(0, 0)

    @pl.loop(0, _CHUNKS, step=2)
    def _chunk_loop(c):
        for s in range(2):
            cc = c + s

            @pl.when(cc + 1 < _CHUNKS)
            def _():
                start_gathers(cc + 1, 1 - s)

            wait_gathers(s)
            accumulate_and_store(cc, s)


def _gather_sum(P, xp):
    mesh = plsc.VectorSubcoreMesh(
        core_axis_name="c", subcore_axis_name="s",
        num_cores=_NC, num_subcores=_NS,
    )
    f = pl.kernel(
        _gather_body,
        out_type=jax.ShapeDtypeStruct((_B, _D), jnp.float32),
        mesh=mesh,
        scratch_types=[
            pltpu.VMEM((_ROWS_PER_W // 2, 2 * _S), jnp.int32),
            pltpu.VMEM((2, _NB * _S, _D), jnp.bfloat16),
            pltpu.VMEM((_NB, _D), jnp.float32),
            pltpu.SemaphoreType.DMA,
            pltpu.SemaphoreType.DMA,
        ],
        compiler_params=pltpu.CompilerParams(
            use_tc_tiling_on_sc=False, needs_layout_passes=False,
        ),
    )
    return f(P, xp)


def kernel(x, emb, W, b):
    x = x.astype(jnp.int32)
    emb_bf = emb.astype(jnp.bfloat16)
    s = jnp.arange(_D)
    perm = 32 * (s // 32) + 16 * (s % 2) + (s % 32) // 2
    W_bf = W[perm, :].astype(jnp.bfloat16)
    b2d = b[perm].reshape(1, _D)
    P = _build_table(emb_bf, W_bf, b2d)
    off = (_V * jnp.arange(_S, dtype=jnp.int32))[None, :]
    xp = (x + off).reshape(_B // 2, 2 * _S)
    return _gather_sum(P, xp)


# speedup vs baseline: 2.8205x; 2.8205x over previous
import jax
import jax.numpy as jnp
from jax import lax
from jax.experimental import pallas as pl
from jax.experimental.pallas import tpu as pltpu
from jax.experimental.pallas import tpu_sc as plsc

_B = 16384
_S = 50
_V = 10000
_D = 128

_NC = 2
_NS = 16
_NW = _NC * _NS
_ROWS_PER_W = _B // _NW
_NB = 8
_CHUNKS = _ROWS_PER_W // _NB
_LANES = 16

_G = 1
_SG = _S // _G


def _make_table_body(add_bias):
    def body(emb_ref, w_ref, b_ref, out_ref):
        j = pl.program_id(0)
        p = lax.dot_general(
            emb_ref[...], w_ref[...],
            dimension_numbers=(((1,), (1,)), ((), ())),
            preferred_element_type=jnp.float32,
        )
        if add_bias:
            sel = jnp.where(j == 0, 1.0, 0.0).astype(jnp.float32)
            p = p + b_ref[...] * sel
        out_ref[...] = p

    return body


def _build_table(emb, W, b2d, add_bias):
    return pl.pallas_call(
        _make_table_body(add_bias),
        grid=(_SG,),
        in_specs=[
            pl.BlockSpec((_V, _D), lambda j: (0, 0)),
            pl.BlockSpec((_D, _D), lambda j: (0, j)),
            pl.BlockSpec((1, _D), lambda j: (0, 0)),
        ],
        out_specs=pl.BlockSpec((_V, _D), lambda j: (j, 0)),
        out_shape=jax.ShapeDtypeStruct((_SG * _V, _D), jnp.float32),
    )(emb, W, b2d)


def _make_gather_body(with_prev):
    n_dma = _NB // 2
    rows_per_chunk = _NB * _SG

    half_chunks = _CHUNKS // 2
    half_idx_rows = _ROWS_PER_W // 4

    def body(p_hbm, x_hbm, *rest):
        if with_prev:
            prev_hbm, out_hbm, idx_all, rows_v, out_v, sem0, sem1 = rest
        else:
            out_hbm, idx_all, rows_v, out_v, sem0, sem1 = rest
        wid = lax.axis_index("s") * _NC + lax.axis_index("c")
        row0 = wid * _ROWS_PER_W
        sems = (sem0, sem1)

        def stage_idx(half):
            pltpu.sync_copy(
                x_hbm.at[pl.ds(
                    wid * (_ROWS_PER_W // 2) + half * half_idx_rows,
                    half_idx_rows,
                )],
                idx_all,
            )

        def start_gathers(chunk, slot):
            for g in range(n_dma):
                pltpu.async_copy(
                    p_hbm.at[idx_all.at[(chunk % half_chunks) * n_dma + g]],
                    rows_v.at[slot, pl.ds(g * 2 * _SG, 2 * _SG)],
                    sems[slot],
                )

        def wait_gathers(slot):
            pltpu.make_async_copy(
                p_hbm.at[pl.ds(0, rows_per_chunk)], rows_v.at[slot],
                sems[slot],
            ).wait()

        def accumulate_and_store(chunk, slot):
            if with_prev:
                pltpu.sync_copy(
                    prev_hbm.at[pl.ds(row0 + chunk * _NB, _NB)], out_v
                )
            for r in range(_NB):
                base = r * _SG

                def jstep(j, acc):
                    return tuple(
                        acc[d]
                        + rows_v[slot, base + j, pl.ds(d * _LANES, _LANES)]
                        for d in range(_D // _LANES)
                    )

                if with_prev:
                    acc = tuple(
                        out_v[r, pl.ds(d * _LANES, _LANES)]
                        for d in range(_D // _LANES)
                    )
                else:
                    acc = tuple(
                        jnp.zeros((_LANES,), jnp.float32)
                        for _ in range(_D // _LANES)
                    )
                acc = lax.fori_loop(0, _SG, jstep, acc, unroll=5)
                for d in range(_D // _LANES):
                    out_v[r, pl.ds(d * _LANES, _LANES)] = acc[d]
            pltpu.sync_copy(
                out_v, out_hbm.at[pl.ds(row0 + chunk * _NB, _NB)]
            )

        stage_idx(0)
        start_gathers(0, 0)

        @pl.loop(0, _CHUNKS, step=2)
        def _chunk_loop(c):
            for s in range(2):
                cc = c + s

                @pl.when(cc == half_chunks - 1)
                def _():
                    wait_gathers(s)
                    stage_idx(1)
                    start_gathers(cc + 1, 1 - s)

                @pl.when(cc != half_chunks - 1)
                def _():
                    @pl.when(cc + 1 < _CHUNKS)
                    def _():
                        start_gathers(cc + 1, 1 - s)

                    wait_gathers(s)

                accumulate_and_store(cc, s)

    return body


def _gather_sum(P, xp, prev=None):
    mesh = plsc.VectorSubcoreMesh(
        core_axis_name="c", subcore_axis_name="s",
        num_cores=_NC, num_subcores=_NS,
    )
    f = pl.kernel(
        _make_gather_body(prev is not None),
        out_type=jax.ShapeDtypeStruct((_B, _D), jnp.float32),
        mesh=mesh,
        scratch_types=[
            pltpu.VMEM((_ROWS_PER_W // 4, 2 * _SG), jnp.int32),
            pltpu.VMEM((2, _NB * _SG, _D), jnp.float32),
            pltpu.VMEM((_NB, _D), jnp.float32),
            pltpu.SemaphoreType.DMA,
            pltpu.SemaphoreType.DMA,
        ],
    )
    args = (P, xp) if prev is None else (P, xp, prev)
    return f(*args)


def kernel(x, emb, W, b):
    x = x.astype(jnp.int32)
    emb_bf = emb.astype(jnp.bfloat16)
    W_bf = W.astype(jnp.bfloat16)
    b2d = b.reshape(1, _D)
    off = (_V * jnp.arange(_SG, dtype=jnp.int32))[None, :]
    out = None
    for g in range(_G):
        Pg = _build_table(
            emb_bf, W_bf[:, g * _SG * _D:(g + 1) * _SG * _D], b2d,
            add_bias=(g == 0),
        )
        xg = (x[:, g * _SG:(g + 1) * _SG] + off).reshape(_B // 2, 2 * _SG)
        out = _gather_sum(Pg, xg, prev=out)
    return out
